# baseline (speedup 1.0000x reference)
import functools

import jax
import jax.numpy as jnp
from jax import lax
from jax.experimental import pallas as pl
from jax.experimental.pallas import tpu as pltpu
from jax.experimental.pallas import tpu_sc as plsc

BATCH = 16384
DIM = 32
NSLOT = 8
NC, NS = 2, 16
NW = NC * NS
BPW = BATCH // NW
CHUNK = 128
NCHUNK = BPW // CHUNK

_mesh = plsc.VectorSubcoreMesh(core_axis_name="c", subcore_axis_name="s")


@functools.partial(
    pl.kernel,
    out_type=jax.ShapeDtypeStruct((BATCH, NSLOT * DIM), jnp.float32),
    mesh=_mesh,
    scratch_types=[
        pltpu.VMEM((NCHUNK, CHUNK), jnp.int32),
        pltpu.VMEM((BPW, DIM), jnp.float32),
        pltpu.SemaphoreType.DMA,
    ],
    compiler_params=pltpu.CompilerParams(use_tc_tiling_on_sc=False),
)
def _gather_concat(idx_hbm, paper_hbm, pfield_hbm, author_hbm, year_hbm,
                   oa_hbm, out_hbm, idx_v, rows_v, sem):
    wid = lax.axis_index("s") * NC + lax.axis_index("c")
    base = wid * BPW
    tables = (paper_hbm, pfield_hbm, pfield_hbm, author_hbm, author_hbm,
              author_hbm, year_hbm, oa_hbm)
    for s, tab in enumerate(tables):
        pltpu.sync_copy(idx_hbm.at[s, wid], idx_v)
        copies = [
            pltpu.async_copy(tab.at[idx_v.at[j]],
                             rows_v.at[pl.ds(j * CHUNK, CHUNK)], sem)
            for j in range(NCHUNK)
        ]
        for c in copies:
            c.wait()
        pltpu.sync_copy(rows_v,
                        out_hbm.at[pl.ds(base, BPW), pl.ds(s * DIM, DIM)])


def kernel(paperId, fieldsOfStudy_0, fieldsOfStudy_1, authors_0, authors_1,
           authors_2, year, isOpenAccess, paper_table, pfield_table,
           author_table, year_table, oa_table):
    idx = jnp.stack([paperId, fieldsOfStudy_0, fieldsOfStudy_1, authors_0,
                     authors_1, authors_2, year, isOpenAccess])
    idx = idx.astype(jnp.int32).reshape(NSLOT, NW, NCHUNK, CHUNK)
    return _gather_concat(idx, paper_table, pfield_table, author_table,
                          year_table, oa_table)
